# baseline (speedup 1.0000x reference)
import functools

import jax
import jax.numpy as jnp
from jax import lax
from jax.experimental import pallas as pl
from jax.experimental.pallas import tpu as pltpu
from jax.experimental.pallas import tpu_sc as plsc

SEQ = 200
BATCH = 4096
EMB = 64
NC = 2
NS = 16
NW = NC * NS
BPW = BATCH // NW
LANES = 16
ESL = EMB // LANES


def _sc_pool_sum(docs_hbm, emb_hbm, out_hbm, idx_v, buf0, buf1, acc, sem0, sem1):
    c = lax.axis_index("c")
    s = lax.axis_index("s")
    wid = c * NS + s
    base = wid * BPW

    pltpu.sync_copy(docs_hbm.at[:, pl.ds(base, BPW)], idx_v)

    zeros = jnp.zeros((LANES,), jnp.float32)

    def zero_row(r, carry):
        for e in range(ESL):
            acc[r, pl.ds(e * LANES, LANES)] = zeros
        return carry

    lax.fori_loop(0, BPW, zero_row, 0, unroll=4)

    bufs = (buf0, buf1)
    sems = (sem0, sem1)

    def start(step, b):
        pltpu.async_copy(emb_hbm.at[idx_v.at[step]], bufs[b], sems[b])

    def wait(b):
        pltpu.make_async_copy(emb_hbm.at[idx_v.at[0]], bufs[b], sems[b]).wait()

    def accum(b):
        buf = bufs[b]

        def row(r, carry):
            for e in range(ESL):
                plsc.addupdate(acc.at[r, pl.ds(e * LANES, LANES)],
                               buf[r, pl.ds(e * LANES, LANES)])
            return carry

        lax.fori_loop(0, BPW, row, 0, unroll=4)

    start(0, 0)
    start(1, 1)

    def body(g, carry):
        for b in range(2):
            step = 2 * g + b
            wait(b)

            @pl.when(step + 2 < SEQ)
            def _():
                start(step + 2, b)

            accum(b)
        return carry

    lax.fori_loop(0, SEQ // 2, body, 0)

    pltpu.sync_copy(acc, out_hbm.at[pl.ds(base, BPW)])


def _pool_sum_sc(docs, emb_table):
    mesh = plsc.VectorSubcoreMesh(core_axis_name="c", subcore_axis_name="s")
    fn = pl.kernel(
        _sc_pool_sum,
        out_type=jax.ShapeDtypeStruct((BATCH, EMB), jnp.float32),
        mesh=mesh,
        scratch_types=[
            pltpu.VMEM((SEQ, BPW), jnp.int32),
            pltpu.VMEM((BPW, EMB), jnp.float32),
            pltpu.VMEM((BPW, EMB), jnp.float32),
            pltpu.VMEM((BPW, EMB), jnp.float32),
            pltpu.SemaphoreType.DMA,
            pltpu.SemaphoreType.DMA,
        ],
        compiler_params=pltpu.CompilerParams(use_tc_tiling_on_sc=False),
    )
    return fn(docs, emb_table)


def _mlp_body(pool_ref, w1_ref, b1_ref, w2_ref, b2_ref, out_ref):
    x = pool_ref[...] * (1.0 / SEQ)
    h = lax.dot_general(x, w1_ref[...], (((1,), (1,)), ((), ())),
                        preferred_element_type=jnp.float32)
    h = jnp.maximum(h + b1_ref[...], 0.0)
    o = lax.dot_general(h, w2_ref[...], (((1,), (1,)), ((), ())),
                        preferred_element_type=jnp.float32)
    out_ref[...] = o + b2_ref[...]


def _mlp_tc(pool_sum, W1, b1, W2, b2):
    return pl.pallas_call(
        _mlp_body,
        out_shape=jax.ShapeDtypeStruct((BATCH, W2.shape[0]), jnp.float32),
    )(pool_sum, W1, b1.reshape(1, -1), W2, b2.reshape(1, -1))


@jax.jit
def kernel(docs, emb_table, W1, b1, W2, b2):
    pool_sum = _pool_sum_sc(docs, emb_table)
    return _mlp_tc(pool_sum, W1, b1, W2, b2)
